# baseline (speedup 1.0000x reference)
import jax
import jax.numpy as jnp
from jax import lax
from jax.experimental import pallas as pl
from jax.experimental.pallas import tpu as pltpu

ROWS = 128
COLS = 100000
BLK_A = 2048
BLK_W = 4096


def _threefry_bits(p):
    ks0 = jnp.uint32(0)
    ks1 = jnp.uint32(42)
    ks2 = ks0 ^ ks1 ^ jnp.uint32(0x1BD11BDA)
    ks = (ks0, ks1, ks2)
    rots = ((13, 15, 26, 6), (17, 29, 16, 24))
    x0 = jnp.full_like(p, ks0)
    x1 = p + ks1
    for i in range(5):
        for r in rots[i % 2]:
            x0 = x0 + x1
            x1 = (x1 << jnp.uint32(r)) | (x1 >> jnp.uint32(32 - r))
            x1 = x0 ^ x1
        x0 = x0 + ks[(i + 1) % 3]
        x1 = x1 + ks[(i + 2) % 3] + jnp.uint32(i + 1)
    return x0 ^ x1


def _gumbel(p):
    bits = _threefry_bits(p)
    fb = (bits >> jnp.uint32(9)) | jnp.uint32(0x3F800000)
    f = lax.bitcast_convert_type(fb, jnp.float32) - jnp.float32(1.0)
    minv = jnp.float32(1e-20)
    maxv = jnp.float32(1.0)
    u = jnp.maximum(minv, f * (maxv - minv) + minv)
    return -jnp.log(-jnp.log(u))


def _argmax_kernel(x_ref, idx_ref, max_s, idx_s):
    j = pl.program_id(0)
    nb = pl.num_programs(0)

    @pl.when(j == 0)
    def _():
        max_s[...] = jnp.full((ROWS, 1), -jnp.inf, jnp.float32)
        idx_s[...] = jnp.zeros((ROWS, 1), jnp.int32)

    v = x_ref[...]
    rows = lax.broadcasted_iota(jnp.uint32, v.shape, 0)
    cols_i = lax.broadcasted_iota(jnp.int32, v.shape, 1) + j * BLK_A
    p = rows * jnp.uint32(COLS) + cols_i.astype(jnp.uint32)
    val = v + _gumbel(p)
    val = jnp.where(cols_i < COLS, val, -jnp.inf)

    m = jnp.max(val, axis=1, keepdims=True)
    cand = jnp.where(val == m, cols_i, jnp.int32(2**31 - 1))
    bi = jnp.min(cand, axis=1, keepdims=True)

    better = m > max_s[...]
    max_s[...] = jnp.where(better, m, max_s[...])
    idx_s[...] = jnp.where(better, bi, idx_s[...])

    @pl.when(j == nb - 1)
    def _():
        idx_ref[...] = idx_s[...]


def _onehot_kernel(idx_ref, o_ref):
    j = pl.program_id(0)
    cols = lax.broadcasted_iota(jnp.int32, o_ref.shape, 1) + j * BLK_W
    o_ref[...] = (cols == idx_ref[...]).astype(jnp.float32)


def kernel(x):
    idx = pl.pallas_call(
        _argmax_kernel,
        grid=(pl.cdiv(COLS, BLK_A),),
        in_specs=[pl.BlockSpec((ROWS, BLK_A), lambda j: (0, j))],
        out_specs=pl.BlockSpec((ROWS, 1), lambda j: (0, 0)),
        out_shape=jax.ShapeDtypeStruct((ROWS, 1), jnp.int32),
        scratch_shapes=[
            pltpu.VMEM((ROWS, 1), jnp.float32),
            pltpu.VMEM((ROWS, 1), jnp.int32),
        ],
    )(x)
    out = pl.pallas_call(
        _onehot_kernel,
        grid=(pl.cdiv(COLS, BLK_W),),
        in_specs=[pl.BlockSpec((ROWS, 1), lambda j: (0, 0))],
        out_specs=pl.BlockSpec((ROWS, BLK_W), lambda j: (0, j)),
        out_shape=jax.ShapeDtypeStruct((ROWS, COLS), jnp.float32),
    )(idx)
    return out
